# baseline (speedup 1.0000x reference)
import functools

import jax
import jax.numpy as jnp
from jax import lax
from jax.experimental import pallas as pl
from jax.experimental.pallas import tpu as pltpu
from jax.experimental.pallas import tpu_sc as plsc

_B, _N, _D = 2, 4096, 4096
_L = 16
_CHUNK = 128
_CPR = _D // _CHUNK
_TOT = _B * _CPR
_NW = _TOT // _L
_NC = 2


def _sc_gather(xr, idxv):
    mesh = plsc.VectorSubcoreMesh(core_axis_name="c", subcore_axis_name="s")

    @functools.partial(
        pl.kernel,
        mesh=mesh,
        out_type=jax.ShapeDtypeStruct((_TOT, _CHUNK), jnp.float32),
        scratch_types=[
            pltpu.VMEM((_L,), jnp.int32),
            pltpu.VMEM((_L, _CHUNK), jnp.float32),
            pltpu.SemaphoreType.DMA,
        ],
    )
    def k(x_hbm, idx_hbm, out_hbm, idx_v, rows_v, sem):
        w = lax.axis_index("s") * _NC + lax.axis_index("c")

        @pl.when(w < _NW)
        def _():
            pltpu.sync_copy(idx_hbm, idx_v)
            vidx = idx_v[...]
            g = lax.iota(jnp.int32, _L) + w * _L
            b1 = g >> 5
            c = g & (_CPR - 1)
            rows = vidx * _CPR + c + b1 * (_N * _CPR)
            pltpu.async_copy(x_hbm.at[rows], rows_v, sem).wait()
            pltpu.sync_copy(rows_v, out_hbm.at[pl.ds(w * _L, _L)])

    return k(xr, idxv)


def kernel(x, indices):
    xr = x.reshape(_B * _N * _CPR, _CHUNK)
    idxv = jnp.full((_L,), indices, dtype=jnp.int32)
    out = _sc_gather(xr, idxv)
    return out.reshape(_B, _D)


# speedup vs baseline: 8.1677x; 8.1677x over previous
import functools

import jax
import jax.numpy as jnp
from jax import lax
from jax.experimental import pallas as pl
from jax.experimental.pallas import tpu as pltpu
from jax.experimental.pallas import tpu_sc as plsc

_B, _N, _D = 2, 4096, 4096
_L = 16
_NC = 2


def _sc_gather(xr, idxv):
    mesh = plsc.VectorSubcoreMesh(core_axis_name="c", subcore_axis_name="s")

    @functools.partial(
        pl.kernel,
        mesh=mesh,
        out_type=jax.ShapeDtypeStruct((_B, _D), jnp.float32),
        scratch_types=[
            pltpu.VMEM((_L,), jnp.int32),
            pltpu.VMEM((_L,), jnp.int32),
            pltpu.VMEM((_B, _D), jnp.float32),
            pltpu.SemaphoreType.DMA,
        ],
    )
    def k(x_hbm, idx_hbm, out_hbm, idx_v, row_v, dst_v, sem):
        w = lax.axis_index("s") * _NC + lax.axis_index("c")

        @pl.when(w == 0)
        def _():
            pltpu.sync_copy(idx_hbm, idx_v)
            row_v[...] = idx_v[...] + lax.iota(jnp.int32, _L) * _N
            pltpu.async_copy(
                x_hbm.at[row_v.at[pl.ds(0, _B)]], dst_v, sem).wait()
            pltpu.sync_copy(dst_v, out_hbm)

    return k(xr, idxv)


def kernel(x, indices):
    xr = x.reshape(_B * _N, _D)
    idxv = jnp.full((_L,), indices, dtype=jnp.int32)
    return _sc_gather(xr, idxv)


# speedup vs baseline: 9.3149x; 1.1405x over previous
import functools

import jax
import jax.numpy as jnp
from jax import lax
from jax.experimental import pallas as pl
from jax.experimental.pallas import tpu as pltpu
from jax.experimental.pallas import tpu_sc as plsc

_B, _N, _D = 2, 4096, 4096
_L = 16


def _sc_slice(xr, idxv):
    mesh = plsc.ScalarSubcoreMesh(axis_name="c", num_cores=1)

    @functools.partial(
        pl.kernel,
        mesh=mesh,
        out_type=jax.ShapeDtypeStruct((_B, _D), jnp.float32),
        scratch_types=[
            pltpu.SMEM((_L,), jnp.int32),
            pltpu.SemaphoreType.DMA,
            pltpu.SemaphoreType.DMA,
        ],
    )
    def k(x_hbm, idx_hbm, out_hbm, idx_s, sem0, sem1):
        pltpu.sync_copy(idx_hbm, idx_s)
        i = idx_s[0]
        c0 = pltpu.async_copy(
            x_hbm.at[pl.ds(i, 1)], out_hbm.at[pl.ds(0, 1)], sem0)
        c1 = pltpu.async_copy(
            x_hbm.at[pl.ds(i + _N, 1)], out_hbm.at[pl.ds(1, 1)], sem1)
        c0.wait()
        c1.wait()

    return k(xr, idxv)


def kernel(x, indices):
    xr = x.reshape(_B * _N, _D)
    idxv = jnp.full((_L,), indices, dtype=jnp.int32)
    return _sc_slice(xr, idxv)


# speedup vs baseline: 79.2624x; 8.5092x over previous
import jax
import jax.numpy as jnp
from jax.experimental import pallas as pl
from jax.experimental.pallas import tpu as pltpu

_B, _N, _D = 2, 4096, 4096
_W = 8


def _slice_body(idx_ref, a_ref, b_ref, o_ref):
    r = idx_ref[0] % _W
    o_ref[0:1, :] = a_ref[pl.ds(r, 1), :]
    o_ref[1:2, :] = b_ref[pl.ds(r, 1), :]


def kernel(x, indices):
    xr = x.reshape(_B * _N, _D)
    idx = jnp.asarray(indices, dtype=jnp.int32).reshape(1)
    grid_spec = pltpu.PrefetchScalarGridSpec(
        num_scalar_prefetch=1,
        grid=(1,),
        in_specs=[
            pl.BlockSpec((_W, _D), lambda i, s: (s[0] // _W, 0)),
            pl.BlockSpec((_W, _D), lambda i, s: (s[0] // _W + _N // _W, 0)),
        ],
        out_specs=pl.BlockSpec((_B, _D), lambda i, s: (0, 0)),
    )
    return pl.pallas_call(
        _slice_body,
        grid_spec=grid_spec,
        out_shape=jax.ShapeDtypeStruct((_B, _D), jnp.float32),
    )(idx, xr, xr)
